# baseline (speedup 1.0000x reference)
import functools

import jax
import jax.numpy as jnp
from jax import lax
from jax.experimental import pallas as pl
from jax.experimental.pallas import tpu as pltpu
from jax.experimental.pallas import tpu_sc as plsc

N = 10000
E = 320000
D = 128
H = 32
EPS = 1e-5

NC = 2
NS = 16
NW = NC * NS
EPW = E // NW
CHUNK = 128
NFULL = EPW // CHUNK
REM = EPW - NFULL * CHUNK
HALF = NFULL // 2
NA = 10112
ZROWS = NA // NS


def _sc_segment_sum(y, ei, zeros):
    mesh = plsc.VectorSubcoreMesh(core_axis_name="c", subcore_axis_name="s")

    @functools.partial(
        pl.kernel,
        mesh=mesh,
        out_type=jax.ShapeDtypeStruct((NC, NA, H), jnp.float32),
        compiler_params=pltpu.CompilerParams(use_tc_tiling_on_sc=False),
        scratch_types=[
            pltpu.VMEM((EPW,), jnp.int32),
            pltpu.VMEM((NFULL, CHUNK), jnp.int32),
            pltpu.VMEM((REM,), jnp.int32),
            pltpu.VMEM((CHUNK, H), jnp.float32),
            pltpu.VMEM((CHUNK, H), jnp.float32),
            pltpu.VMEM((REM, H), jnp.float32),
            pltpu.VMEM_SHARED((NA, H), jnp.float32),
            pltpu.SemaphoreType.DMA,
            pltpu.SemaphoreType.DMA,
            pltpu.SemaphoreType.DMA,
            pltpu.SemaphoreType.DMA,
        ],
    )
    def k(y_hbm, ei_hbm, z_hbm, out_hbm, src_v, dst_v, dstr_v,
          rows_a, rows_b, rows_r, acc, semz, semi, sem_a, sem_b):
        cid = lax.axis_index("c")
        sid = lax.axis_index("s")
        wid = cid * NS + sid
        wb = wid * EPW
        zdesc = pltpu.make_async_copy(z_hbm.at[pl.ds(sid * ZROWS, ZROWS)],
                                      acc.at[pl.ds(sid * ZROWS, ZROWS)], semz)
        zdesc.start()
        sdesc = pltpu.make_async_copy(ei_hbm.at[0, pl.ds(wb, EPW)], src_v, semi)
        sdesc.start()

        def dstage(c, carry):
            pltpu.async_copy(ei_hbm.at[1, pl.ds(wb + c * CHUNK, CHUNK)],
                             dst_v.at[c], semi)
            return carry

        lax.fori_loop(0, NFULL, dstage, 0)
        rdesc = pltpu.make_async_copy(
            ei_hbm.at[1, pl.ds(wb + NFULL * CHUNK, REM)], dstr_v, semi)
        rdesc.start()
        sdesc.wait()

        def dwait(c, carry):
            pltpu.make_async_copy(ei_hbm.at[1, pl.ds(wb + c * CHUNK, CHUNK)],
                                  dst_v.at[c], semi).wait()
            return carry

        lax.fori_loop(0, NFULL, dwait, 0)
        rdesc.wait()
        zdesc.wait()
        plsc.subcore_barrier()

        def gather(c, rows, sem):
            return pltpu.make_async_copy(
                y_hbm.at[src_v.at[pl.ds(c * CHUNK, CHUNK)]], rows, sem)

        gather(0, rows_a, sem_a).start()

        def body(i, carry):
            c0 = 2 * i
            gather(c0 + 1, rows_b, sem_b).start()
            gather(c0, rows_a, sem_a).wait()
            pltpu.sync_copy(rows_a, acc.at[dst_v.at[c0]], add=True)

            @pl.when(i < HALF - 1)
            def _start_next():
                gather(c0 + 2, rows_a, sem_a).start()

            gather(c0 + 1, rows_b, sem_b).wait()
            pltpu.sync_copy(rows_b, acc.at[dst_v.at[c0 + 1]], add=True)
            return carry

        lax.fori_loop(0, HALF, body, 0)
        pltpu.async_copy(y_hbm.at[src_v.at[pl.ds(NFULL * CHUNK, REM)]],
                         rows_r, sem_a).wait()
        pltpu.sync_copy(rows_r, acc.at[dstr_v], add=True)
        plsc.subcore_barrier()
        pltpu.sync_copy(acc.at[pl.ds(sid * ZROWS, ZROWS)],
                        out_hbm.at[cid, pl.ds(sid * ZROWS, ZROWS)])

    return k(y, ei, zeros)[:, :N]


def _tc_in(x, W1l, W1r):
    def body(x_ref, wl_ref, wr_ref, y_ref, r_ref):
        xv = x_ref[...]
        y_ref[...] = jnp.dot(xv, wl_ref[...], preferred_element_type=jnp.float32, precision=lax.Precision.HIGHEST)
        r_ref[...] = jnp.dot(xv, wr_ref[...], preferred_element_type=jnp.float32, precision=lax.Precision.HIGHEST)

    return pl.pallas_call(
        body,
        out_shape=[jax.ShapeDtypeStruct((N, H), jnp.float32),
                   jax.ShapeDtypeStruct((N, H), jnp.float32)],
    )(x, W1l, W1r)


def _rsqrt(a):
    r = lax.rsqrt(a)
    r = r * (1.5 - 0.5 * a * r * r)
    r = r * (1.5 - 0.5 * a * r * r)
    return r


def _post(h, g, be):
    m = jnp.mean(h, axis=0, keepdims=True)
    v = jnp.mean((h - m) ** 2, axis=0, keepdims=True)
    h = (h - m) * _rsqrt(v + EPS) * g + be
    h = jnp.where(h >= 0, h, 0.01 * h)
    s = jnp.maximum(jnp.sum(h * h, axis=-1, keepdims=True), 1e-24)
    return h * _rsqrt(s)


def _tc_mid(p, r, bl, g, be, Wl, Wr):
    def body(p_ref, r_ref, bl_ref, g_ref, be_ref, wl_ref, wr_ref, y_ref, ro_ref):
        h = p_ref[0] + p_ref[1] + r_ref[...] + bl_ref[...]
        h = _post(h, g_ref[...], be_ref[...])
        y_ref[...] = jnp.dot(h, wl_ref[...], preferred_element_type=jnp.float32, precision=lax.Precision.HIGHEST)
        ro_ref[...] = jnp.dot(h, wr_ref[...], preferred_element_type=jnp.float32, precision=lax.Precision.HIGHEST)

    return pl.pallas_call(
        body,
        out_shape=[jax.ShapeDtypeStruct((N, H), jnp.float32),
                   jax.ShapeDtypeStruct((N, H), jnp.float32)],
    )(p, r, bl, g, be, Wl, Wr)


def _tc_out(p, r, bl, g, be, Wp, bp):
    def body(p_ref, r_ref, bl_ref, g_ref, be_ref, wp_ref, bp_ref, o_ref):
        h = p_ref[0] + p_ref[1] + r_ref[...] + bl_ref[...]
        h = _post(h, g_ref[...], be_ref[...])
        o_ref[...] = (jnp.dot(h, wp_ref[...], preferred_element_type=jnp.float32, precision=lax.Precision.HIGHEST)
                      + bp_ref[...])

    return pl.pallas_call(
        body,
        out_shape=jax.ShapeDtypeStruct((N, H), jnp.float32),
    )(p, r, bl, g, be, Wp, bp)


def kernel(x, edge_index, W1l, b1l, W1r, g1, be1, W2l, b2l, W2r, g2, be2, Wp, bp):
    zeros = jnp.zeros((NA, H), jnp.float32)

    b1l_, g1_, be1_ = b1l.reshape(1, H), g1.reshape(1, H), be1.reshape(1, H)
    b2l_, g2_, be2_ = b2l.reshape(1, H), g2.reshape(1, H), be2.reshape(1, H)
    bp_ = bp.reshape(1, H)

    y1, r1 = _tc_in(x, W1l, W1r)
    p1 = _sc_segment_sum(y1, edge_index, zeros)
    y2, r2 = _tc_mid(p1, r1, b1l_, g1_, be1_, W2l, W2r)
    p2 = _sc_segment_sum(y2, edge_index, zeros)
    return _tc_out(p2, r2, b2l_, g2_, be2_, Wp, bp_)


# speedup vs baseline: 1.1211x; 1.1211x over previous
import functools

import jax
import jax.numpy as jnp
from jax import lax
from jax.experimental import pallas as pl
from jax.experimental.pallas import tpu as pltpu
from jax.experimental.pallas import tpu_sc as plsc

N = 10000
E = 320000
D = 128
H = 32
EPS = 1e-5

NC = 2
NS = 16
NW = NC * NS
EPW = E // NW
CHUNK = 128
NFULL = EPW // CHUNK
REM = EPW - NFULL * CHUNK
HALF = NFULL // 2
NA = 10112
ZROWS = NA // NS


def _sc_segment_sum(y, ei, zeros):
    mesh = plsc.VectorSubcoreMesh(core_axis_name="c", subcore_axis_name="s")

    @functools.partial(
        pl.kernel,
        mesh=mesh,
        out_type=jax.ShapeDtypeStruct((NC, NA, H), jnp.float32),
        compiler_params=pltpu.CompilerParams(use_tc_tiling_on_sc=False),
        scratch_types=[
            pltpu.VMEM((EPW,), jnp.int32),
            pltpu.VMEM((NFULL, CHUNK), jnp.int32),
            pltpu.VMEM((REM,), jnp.int32),
            pltpu.VMEM((CHUNK, H), jnp.float32),
            pltpu.VMEM((CHUNK, H), jnp.float32),
            pltpu.VMEM((REM, H), jnp.float32),
            pltpu.VMEM_SHARED((NA, H), jnp.float32),
            pltpu.SemaphoreType.DMA,
            pltpu.SemaphoreType.DMA,
            pltpu.SemaphoreType.DMA,
            pltpu.SemaphoreType.DMA,
        ],
    )
    def k(y_hbm, ei_hbm, z_hbm, out_hbm, src_v, dst_v, dstr_v,
          rows_a, rows_b, rows_r, acc, semz, semi, sem_a, sem_b):
        cid = lax.axis_index("c")
        sid = lax.axis_index("s")
        wid = cid * NS + sid
        wb = wid * EPW
        zdesc = pltpu.make_async_copy(z_hbm.at[pl.ds(sid * ZROWS, ZROWS)],
                                      acc.at[pl.ds(sid * ZROWS, ZROWS)], semz)
        zdesc.start()
        sdesc = pltpu.make_async_copy(ei_hbm.at[0, pl.ds(wb, EPW)], src_v, semi)
        sdesc.start()

        def dstage(c, carry):
            pltpu.async_copy(ei_hbm.at[1, pl.ds(wb + c * CHUNK, CHUNK)],
                             dst_v.at[c], semi)
            return carry

        lax.fori_loop(0, NFULL, dstage, 0)
        rdesc = pltpu.make_async_copy(
            ei_hbm.at[1, pl.ds(wb + NFULL * CHUNK, REM)], dstr_v, semi)
        rdesc.start()
        sdesc.wait()

        def dwait(c, carry):
            pltpu.make_async_copy(ei_hbm.at[1, pl.ds(wb + c * CHUNK, CHUNK)],
                                  dst_v.at[c], semi).wait()
            return carry

        lax.fori_loop(0, NFULL, dwait, 0)
        rdesc.wait()
        zdesc.wait()
        plsc.subcore_barrier()

        def gather(c, rows, sem):
            return pltpu.make_async_copy(
                y_hbm.at[src_v.at[pl.ds(c * CHUNK, CHUNK)]], rows, sem)

        gather(0, rows_a, sem_a).start()

        def body(i, carry):
            c0 = 2 * i
            gather(c0 + 1, rows_b, sem_b).start()
            gather(c0, rows_a, sem_a).wait()
            pltpu.sync_copy(rows_a, acc.at[dst_v.at[c0]], add=True)

            @pl.when(i < HALF - 1)
            def _start_next():
                gather(c0 + 2, rows_a, sem_a).start()

            gather(c0 + 1, rows_b, sem_b).wait()
            pltpu.sync_copy(rows_b, acc.at[dst_v.at[c0 + 1]], add=True)
            return carry

        lax.fori_loop(0, HALF, body, 0)
        pltpu.async_copy(y_hbm.at[src_v.at[pl.ds(NFULL * CHUNK, REM)]],
                         rows_r, sem_a).wait()
        pltpu.sync_copy(rows_r, acc.at[dstr_v], add=True)
        plsc.subcore_barrier()
        pltpu.sync_copy(acc.at[pl.ds(sid * ZROWS, ZROWS)],
                        out_hbm.at[cid, pl.ds(sid * ZROWS, ZROWS)])

    return k(y, ei, zeros)[:, :N]


def _tc_in(x, W1l, W1r):
    def body(x_ref, wl_ref, wr_ref, y_ref, r_ref):
        xv = x_ref[...]
        y_ref[...] = jnp.dot(xv, wl_ref[...], preferred_element_type=jnp.float32)
        r_ref[...] = jnp.dot(xv, wr_ref[...], preferred_element_type=jnp.float32)

    return pl.pallas_call(
        body,
        out_shape=[jax.ShapeDtypeStruct((N, H), jnp.float32),
                   jax.ShapeDtypeStruct((N, H), jnp.float32)],
    )(x, W1l, W1r)


def _rsqrt(a):
    r = lax.rsqrt(a)
    r = r * (1.5 - 0.5 * a * r * r)
    r = r * (1.5 - 0.5 * a * r * r)
    return r


def _post(h, g, be):
    m = jnp.mean(h, axis=0, keepdims=True)
    v = jnp.mean((h - m) ** 2, axis=0, keepdims=True)
    h = (h - m) * _rsqrt(v + EPS) * g + be
    h = jnp.where(h >= 0, h, 0.01 * h)
    s = jnp.maximum(jnp.sum(h * h, axis=-1, keepdims=True), 1e-24)
    return h * _rsqrt(s)


def _tc_mid(p, r, bl, g, be, Wl, Wr):
    def body(p_ref, r_ref, bl_ref, g_ref, be_ref, wl_ref, wr_ref, y_ref, ro_ref):
        h = p_ref[0] + p_ref[1] + r_ref[...] + bl_ref[...]
        h = _post(h, g_ref[...], be_ref[...])
        y_ref[...] = jnp.dot(h, wl_ref[...], preferred_element_type=jnp.float32)
        ro_ref[...] = jnp.dot(h, wr_ref[...], preferred_element_type=jnp.float32)

    return pl.pallas_call(
        body,
        out_shape=[jax.ShapeDtypeStruct((N, H), jnp.float32),
                   jax.ShapeDtypeStruct((N, H), jnp.float32)],
    )(p, r, bl, g, be, Wl, Wr)


def _tc_out(p, r, bl, g, be, Wp, bp):
    def body(p_ref, r_ref, bl_ref, g_ref, be_ref, wp_ref, bp_ref, o_ref):
        h = p_ref[0] + p_ref[1] + r_ref[...] + bl_ref[...]
        h = _post(h, g_ref[...], be_ref[...])
        o_ref[...] = (jnp.dot(h, wp_ref[...], preferred_element_type=jnp.float32)
                      + bp_ref[...])

    return pl.pallas_call(
        body,
        out_shape=jax.ShapeDtypeStruct((N, H), jnp.float32),
    )(p, r, bl, g, be, Wp, bp)


def kernel(x, edge_index, W1l, b1l, W1r, g1, be1, W2l, b2l, W2r, g2, be2, Wp, bp):
    zeros = jnp.zeros((NA, H), jnp.float32)

    b1l_, g1_, be1_ = b1l.reshape(1, H), g1.reshape(1, H), be1.reshape(1, H)
    b2l_, g2_, be2_ = b2l.reshape(1, H), g2.reshape(1, H), be2.reshape(1, H)
    bp_ = bp.reshape(1, H)

    y1, r1 = _tc_in(x, W1l, W1r)
    p1 = _sc_segment_sum(y1, edge_index, zeros)
    y2, r2 = _tc_mid(p1, r1, b1l_, g1_, be1_, W2l, W2r)
    p2 = _sc_segment_sum(y2, edge_index, zeros)
    return _tc_out(p2, r2, b2l_, g2_, be2_, Wp, bp_)


# speedup vs baseline: 1.2915x; 1.1521x over previous
import functools

import jax
import jax.numpy as jnp
from jax import lax
from jax.experimental import pallas as pl
from jax.experimental.pallas import tpu as pltpu
from jax.experimental.pallas import tpu_sc as plsc

N = 10000
E = 320000
D = 128
H = 32
EPS = 1e-5

NC = 2
NS = 16
NW = NC * NS
EPW = E // NW
CHUNK = 128
NFULL = EPW // CHUNK
REM = EPW - NFULL * CHUNK
RING = 4
RITER = NFULL // RING
LEFT = NFULL - RITER * RING
NA = 10112
ZROWS = NA // NS


def _sc_segment_sum(y, ei, zeros):
    mesh = plsc.VectorSubcoreMesh(core_axis_name="c", subcore_axis_name="s")

    @functools.partial(
        pl.kernel,
        mesh=mesh,
        out_type=jax.ShapeDtypeStruct((NC, NA, H), jnp.float32),
        compiler_params=pltpu.CompilerParams(use_tc_tiling_on_sc=False),
        scratch_types=[
            pltpu.VMEM((EPW,), jnp.int32),
            pltpu.VMEM((NFULL, CHUNK), jnp.int32),
            pltpu.VMEM((REM,), jnp.int32),
            [pltpu.VMEM((CHUNK, H), jnp.float32)] * RING,
            pltpu.VMEM((REM, H), jnp.float32),
            pltpu.VMEM_SHARED((NA, H), jnp.float32),
            pltpu.SemaphoreType.DMA,
            pltpu.SemaphoreType.DMA,
            pltpu.SemaphoreType.DMA,
            [pltpu.SemaphoreType.DMA] * RING,
            [pltpu.SemaphoreType.DMA] * RING,
        ],
    )
    def k(y_hbm, ei_hbm, z_hbm, out_hbm, src_v, dst_v, dstr_v,
          rows, rows_r, acc, semz, semi, semd, sem_g, sem_s):
        cid = lax.axis_index("c")
        sid = lax.axis_index("s")
        wid = cid * NS + sid
        wb = wid * EPW
        zdesc = pltpu.make_async_copy(z_hbm.at[pl.ds(sid * ZROWS, ZROWS)],
                                      acc.at[pl.ds(sid * ZROWS, ZROWS)], semz)
        zdesc.start()
        sdesc = pltpu.make_async_copy(ei_hbm.at[0, pl.ds(wb, EPW)], src_v, semi)
        sdesc.start()

        def dstage(c, carry):
            pltpu.async_copy(ei_hbm.at[1, pl.ds(wb + c * CHUNK, CHUNK)],
                             dst_v.at[c], semd)
            return carry

        lax.fori_loop(0, NFULL, dstage, 0)
        rdesc = pltpu.make_async_copy(
            ei_hbm.at[1, pl.ds(wb + NFULL * CHUNK, REM)], dstr_v, semd)
        rdesc.start()

        def gather(c, b):
            return pltpu.make_async_copy(
                y_hbm.at[src_v.at[pl.ds(c * CHUNK, CHUNK)]], rows[b], sem_g[b])

        def scatter(c, b):
            return pltpu.make_async_copy(rows[b], acc.at[dst_v.at[c]], sem_s[b])

        sdesc.wait()
        for b in range(RING):
            gather(b, b).start()

        def dwait(c, carry):
            pltpu.make_async_copy(ei_hbm.at[1, pl.ds(wb + c * CHUNK, CHUNK)],
                                  dst_v.at[c], semd).wait()
            return carry

        lax.fori_loop(0, NFULL, dwait, 0)
        rdesc.wait()
        zdesc.wait()
        plsc.subcore_barrier()

        def body(i, carry):
            c0 = RING * i
            for b in range(RING):
                gather(c0 + b, b).wait()
                scatter(c0 + b, b).start(add=True)
            for b in range(LEFT):
                scatter(c0 + b, b).wait()
                gather(c0 + b + RING, b).start()

            @pl.when(i < RITER - 1)
            def _more():
                for b in range(LEFT, RING):
                    scatter(c0 + b, b).wait()
                    gather(c0 + b + RING, b).start()

            return carry

        lax.fori_loop(0, RITER, body, 0)
        cbase = RITER * RING
        for b in range(LEFT):
            gather(cbase + b, b).wait()
            scatter(cbase + b, b).start(add=True)
        pltpu.async_copy(y_hbm.at[src_v.at[pl.ds(NFULL * CHUNK, REM)]],
                         rows_r, sem_g[0]).wait()
        pltpu.sync_copy(rows_r, acc.at[dstr_v], add=True)
        for b in range(LEFT):
            scatter(cbase + b, b).wait()
        for b in range(LEFT, RING):
            scatter(cbase - RING + b, b).wait()
        plsc.subcore_barrier()
        pltpu.sync_copy(acc.at[pl.ds(sid * ZROWS, ZROWS)],
                        out_hbm.at[cid, pl.ds(sid * ZROWS, ZROWS)])

    return k(y, ei, zeros)[:, :N]


def _tc_in(x, W1l, W1r):
    def body(x_ref, wl_ref, wr_ref, y_ref, r_ref):
        xv = x_ref[...]
        y_ref[...] = jnp.dot(xv, wl_ref[...], preferred_element_type=jnp.float32)
        r_ref[...] = jnp.dot(xv, wr_ref[...], preferred_element_type=jnp.float32)

    return pl.pallas_call(
        body,
        out_shape=[jax.ShapeDtypeStruct((N, H), jnp.float32),
                   jax.ShapeDtypeStruct((N, H), jnp.float32)],
    )(x, W1l, W1r)


def _rsqrt(a):
    r = lax.rsqrt(a)
    r = r * (1.5 - 0.5 * a * r * r)
    r = r * (1.5 - 0.5 * a * r * r)
    return r


def _post(h, g, be):
    m = jnp.mean(h, axis=0, keepdims=True)
    v = jnp.mean((h - m) ** 2, axis=0, keepdims=True)
    h = (h - m) * _rsqrt(v + EPS) * g + be
    h = jnp.where(h >= 0, h, 0.01 * h)
    s = jnp.maximum(jnp.sum(h * h, axis=-1, keepdims=True), 1e-24)
    return h * _rsqrt(s)


def _tc_mid(p, r, bl, g, be, Wl, Wr):
    def body(p_ref, r_ref, bl_ref, g_ref, be_ref, wl_ref, wr_ref, y_ref, ro_ref):
        h = p_ref[0] + p_ref[1] + r_ref[...] + bl_ref[...]
        h = _post(h, g_ref[...], be_ref[...])
        y_ref[...] = jnp.dot(h, wl_ref[...], preferred_element_type=jnp.float32)
        ro_ref[...] = jnp.dot(h, wr_ref[...], preferred_element_type=jnp.float32)

    return pl.pallas_call(
        body,
        out_shape=[jax.ShapeDtypeStruct((N, H), jnp.float32),
                   jax.ShapeDtypeStruct((N, H), jnp.float32)],
    )(p, r, bl, g, be, Wl, Wr)


def _tc_out(p, r, bl, g, be, Wp, bp):
    def body(p_ref, r_ref, bl_ref, g_ref, be_ref, wp_ref, bp_ref, o_ref):
        h = p_ref[0] + p_ref[1] + r_ref[...] + bl_ref[...]
        h = _post(h, g_ref[...], be_ref[...])
        o_ref[...] = (jnp.dot(h, wp_ref[...], preferred_element_type=jnp.float32)
                      + bp_ref[...])

    return pl.pallas_call(
        body,
        out_shape=jax.ShapeDtypeStruct((N, H), jnp.float32),
    )(p, r, bl, g, be, Wp, bp)


def kernel(x, edge_index, W1l, b1l, W1r, g1, be1, W2l, b2l, W2r, g2, be2, Wp, bp):
    zeros = jnp.zeros((NA, H), jnp.float32)

    b1l_, g1_, be1_ = b1l.reshape(1, H), g1.reshape(1, H), be1.reshape(1, H)
    b2l_, g2_, be2_ = b2l.reshape(1, H), g2.reshape(1, H), be2.reshape(1, H)
    bp_ = bp.reshape(1, H)

    y1, r1 = _tc_in(x, W1l, W1r)
    p1 = _sc_segment_sum(y1, edge_index, zeros)
    y2, r2 = _tc_mid(p1, r1, b1l_, g1_, be1_, W2l, W2r)
    p2 = _sc_segment_sum(y2, edge_index, zeros)
    return _tc_out(p2, r2, b2l_, g2_, be2_, Wp, bp_)


# speedup vs baseline: 1.4673x; 1.1361x over previous
import functools

import jax
import jax.numpy as jnp
from jax import lax
from jax.experimental import pallas as pl
from jax.experimental.pallas import tpu as pltpu
from jax.experimental.pallas import tpu_sc as plsc

N = 10000
E = 320000
D = 128
H = 32
EPS = 1e-5

NC = 2
NS = 16
NW = NC * NS
EPW = E // NW
CHUNK = 128
NFULL = EPW // CHUNK
REM = EPW - NFULL * CHUNK
RING = 6
RITER = NFULL // RING
LEFT = NFULL - RITER * RING
NA = 10112
ZROWS = NA // NS


def _sc_segment_sum(y, ei, zeros):
    mesh = plsc.VectorSubcoreMesh(core_axis_name="c", subcore_axis_name="s")

    @functools.partial(
        pl.kernel,
        mesh=mesh,
        out_type=jax.ShapeDtypeStruct((NC, NA, H), jnp.float32),
        compiler_params=pltpu.CompilerParams(use_tc_tiling_on_sc=False),
        scratch_types=[
            pltpu.VMEM((EPW,), jnp.int32),
            pltpu.VMEM((NFULL, CHUNK), jnp.int32),
            pltpu.VMEM((REM,), jnp.int32),
            [pltpu.VMEM((CHUNK, H), jnp.float32)] * RING,
            pltpu.VMEM((REM, H), jnp.float32),
            pltpu.VMEM_SHARED((NA, H), jnp.float32),
            pltpu.SemaphoreType.DMA,
            pltpu.SemaphoreType.DMA,
            pltpu.SemaphoreType.DMA,
            [pltpu.SemaphoreType.DMA] * RING,
            [pltpu.SemaphoreType.DMA] * RING,
        ],
    )
    def k(y_hbm, ei_hbm, z_hbm, out_hbm, src_v, dst_v, dstr_v,
          rows, rows_r, acc, semz, semi, semd, sem_g, sem_s):
        cid = lax.axis_index("c")
        sid = lax.axis_index("s")
        wid = cid * NS + sid
        wb = wid * EPW
        zdesc = pltpu.make_async_copy(z_hbm.at[pl.ds(sid * ZROWS, ZROWS)],
                                      acc.at[pl.ds(sid * ZROWS, ZROWS)], semz)
        zdesc.start()
        sdesc = pltpu.make_async_copy(ei_hbm.at[0, pl.ds(wb, EPW)], src_v, semi)
        sdesc.start()

        def dstage(c, carry):
            pltpu.async_copy(ei_hbm.at[1, pl.ds(wb + c * CHUNK, CHUNK)],
                             dst_v.at[c], semd)
            return carry

        lax.fori_loop(0, NFULL, dstage, 0)
        rdesc = pltpu.make_async_copy(
            ei_hbm.at[1, pl.ds(wb + NFULL * CHUNK, REM)], dstr_v, semd)
        rdesc.start()

        def gather(c, b):
            return pltpu.make_async_copy(
                y_hbm.at[src_v.at[pl.ds(c * CHUNK, CHUNK)]], rows[b], sem_g[b])

        def scatter(c, b):
            return pltpu.make_async_copy(rows[b], acc.at[dst_v.at[c]], sem_s[b])

        sdesc.wait()
        for b in range(RING):
            gather(b, b).start()
        rem_g = pltpu.make_async_copy(
            y_hbm.at[src_v.at[pl.ds(NFULL * CHUNK, REM)]], rows_r, semi)
        rem_g.start()

        def dwait(c, carry):
            pltpu.make_async_copy(ei_hbm.at[1, pl.ds(wb + c * CHUNK, CHUNK)],
                                  dst_v.at[c], semd).wait()
            return carry

        lax.fori_loop(0, NFULL, dwait, 0)
        rdesc.wait()
        zdesc.wait()
        plsc.subcore_barrier()

        def body(i, carry):
            c0 = RING * i
            for b in range(RING):
                gather(c0 + b, b).wait()
                scatter(c0 + b, b).start(add=True)
            for b in range(LEFT):
                scatter(c0 + b, b).wait()
                gather(c0 + b + RING, b).start()

            @pl.when(i < RITER - 1)
            def _more():
                for b in range(LEFT, RING):
                    scatter(c0 + b, b).wait()
                    gather(c0 + b + RING, b).start()

            return carry

        lax.fori_loop(0, RITER, body, 0)
        cbase = RITER * RING
        rem_g.wait()
        pltpu.sync_copy(rows_r, acc.at[dstr_v], add=True)
        for b in range(LEFT, RING):
            scatter(cbase - RING + b, b).wait()
        plsc.subcore_barrier()
        pltpu.sync_copy(acc.at[pl.ds(sid * ZROWS, ZROWS)],
                        out_hbm.at[cid, pl.ds(sid * ZROWS, ZROWS)])

    return k(y, ei, zeros)


def _tc_in(x, W1l, W1r):
    def body(x_ref, wl_ref, wr_ref, y_ref, r_ref):
        xv = x_ref[...]
        y_ref[...] = jnp.dot(xv, wl_ref[...], preferred_element_type=jnp.float32)
        r_ref[...] = jnp.dot(xv, wr_ref[...], preferred_element_type=jnp.float32)

    return pl.pallas_call(
        body,
        out_shape=[jax.ShapeDtypeStruct((N, H), jnp.float32),
                   jax.ShapeDtypeStruct((N, H), jnp.float32)],
    )(x, W1l, W1r)


def _rsqrt(a):
    r = lax.rsqrt(a)
    r = r * (1.5 - 0.5 * a * r * r)
    r = r * (1.5 - 0.5 * a * r * r)
    return r


def _post(h, g, be):
    m = jnp.mean(h, axis=0, keepdims=True)
    v = jnp.mean((h - m) ** 2, axis=0, keepdims=True)
    h = (h - m) * _rsqrt(v + EPS) * g + be
    h = jnp.where(h >= 0, h, 0.01 * h)
    s = jnp.maximum(jnp.sum(h * h, axis=-1, keepdims=True), 1e-24)
    return h * _rsqrt(s)


def _tc_mid(p, r, bl, g, be, Wl, Wr):
    def body(p_ref, r_ref, bl_ref, g_ref, be_ref, wl_ref, wr_ref, y_ref, ro_ref):
        h = p_ref[0, :N] + p_ref[1, :N] + r_ref[...] + bl_ref[...]
        h = _post(h, g_ref[...], be_ref[...])
        y_ref[...] = jnp.dot(h, wl_ref[...], preferred_element_type=jnp.float32)
        ro_ref[...] = jnp.dot(h, wr_ref[...], preferred_element_type=jnp.float32)

    return pl.pallas_call(
        body,
        out_shape=[jax.ShapeDtypeStruct((N, H), jnp.float32),
                   jax.ShapeDtypeStruct((N, H), jnp.float32)],
    )(p, r, bl, g, be, Wl, Wr)


def _tc_out(p, r, bl, g, be, Wp, bp):
    def body(p_ref, r_ref, bl_ref, g_ref, be_ref, wp_ref, bp_ref, o_ref):
        h = p_ref[0, :N] + p_ref[1, :N] + r_ref[...] + bl_ref[...]
        h = _post(h, g_ref[...], be_ref[...])
        o_ref[...] = (jnp.dot(h, wp_ref[...], preferred_element_type=jnp.float32)
                      + bp_ref[...])

    return pl.pallas_call(
        body,
        out_shape=jax.ShapeDtypeStruct((N, H), jnp.float32),
    )(p, r, bl, g, be, Wp, bp)


def kernel(x, edge_index, W1l, b1l, W1r, g1, be1, W2l, b2l, W2r, g2, be2, Wp, bp):
    zeros = jnp.zeros((NA, H), jnp.float32)

    b1l_, g1_, be1_ = b1l.reshape(1, H), g1.reshape(1, H), be1.reshape(1, H)
    b2l_, g2_, be2_ = b2l.reshape(1, H), g2.reshape(1, H), be2.reshape(1, H)
    bp_ = bp.reshape(1, H)

    y1, r1 = _tc_in(x, W1l, W1r)
    p1 = _sc_segment_sum(y1, edge_index, zeros)
    y2, r2 = _tc_mid(p1, r1, b1l_, g1_, be1_, W2l, W2r)
    p2 = _sc_segment_sum(y2, edge_index, zeros)
    return _tc_out(p2, r2, b2l_, g2_, be2_, Wp, bp_)


# speedup vs baseline: 1.4717x; 1.0030x over previous
import functools

import jax
import jax.numpy as jnp
from jax import lax
from jax.experimental import pallas as pl
from jax.experimental.pallas import tpu as pltpu
from jax.experimental.pallas import tpu_sc as plsc

N = 10000
E = 320000
D = 128
H = 32
EPS = 1e-5

NC = 2
NS = 16
NW = NC * NS
EPW = E // NW
CHUNK = 128
NFULL = EPW // CHUNK
REM = EPW - NFULL * CHUNK
RING = 6
RITER = NFULL // RING
LEFT = NFULL - RITER * RING
NA = 10112
ZROWS = NA // NS


def _sc_segment_sum(y, ei, zeros):
    mesh = plsc.VectorSubcoreMesh(core_axis_name="c", subcore_axis_name="s")

    @functools.partial(
        pl.kernel,
        mesh=mesh,
        out_type=jax.ShapeDtypeStruct((NC, NA, H), jnp.float32),
        compiler_params=pltpu.CompilerParams(use_tc_tiling_on_sc=False),
        scratch_types=[
            pltpu.VMEM((EPW,), jnp.int32),
            pltpu.VMEM((NFULL, CHUNK), jnp.int32),
            pltpu.VMEM((REM,), jnp.int32),
            [pltpu.VMEM((CHUNK, H), jnp.float32)] * RING,
            pltpu.VMEM((REM, H), jnp.float32),
            pltpu.VMEM_SHARED((NA, H), jnp.float32),
            pltpu.SemaphoreType.DMA,
            pltpu.SemaphoreType.DMA,
            pltpu.SemaphoreType.DMA,
            [pltpu.SemaphoreType.DMA] * RING,
            [pltpu.SemaphoreType.DMA] * RING,
        ],
    )
    def k(y_hbm, ei_hbm, z_hbm, out_hbm, src_v, dst_v, dstr_v,
          rows, rows_r, acc, semz, semi, semd, sem_g, sem_s):
        cid = lax.axis_index("c")
        sid = lax.axis_index("s")
        wid = cid * NS + sid
        wb = wid * EPW
        zdesc = pltpu.make_async_copy(z_hbm.at[pl.ds(sid * ZROWS, ZROWS)],
                                      acc.at[pl.ds(sid * ZROWS, ZROWS)], semz)
        zdesc.start()
        sdesc = pltpu.make_async_copy(ei_hbm.at[0, pl.ds(wb, EPW)], src_v, semi)
        sdesc.start()

        def dstage(c, carry):
            pltpu.async_copy(ei_hbm.at[1, pl.ds(wb + c * CHUNK, CHUNK)],
                             dst_v.at[c], semd)
            return carry

        lax.fori_loop(0, NFULL, dstage, 0)
        rdesc = pltpu.make_async_copy(
            ei_hbm.at[1, pl.ds(wb + NFULL * CHUNK, REM)], dstr_v, semd)
        rdesc.start()

        def gather(c, b):
            return pltpu.make_async_copy(
                y_hbm.at[src_v.at[pl.ds(c * CHUNK, CHUNK)]], rows[b], sem_g[b])

        def scatter(c, b):
            return pltpu.make_async_copy(rows[b], acc.at[dst_v.at[c]], sem_s[b])

        sdesc.wait()
        for b in range(RING):
            gather(b, b).start()
        rem_g = pltpu.make_async_copy(
            y_hbm.at[src_v.at[pl.ds(NFULL * CHUNK, REM)]], rows_r, semi)
        rem_g.start()

        def dwait(c, carry):
            pltpu.make_async_copy(ei_hbm.at[1, pl.ds(wb + c * CHUNK, CHUNK)],
                                  dst_v.at[c], semd).wait()
            return carry

        lax.fori_loop(0, NFULL, dwait, 0)
        rdesc.wait()
        zdesc.wait()
        plsc.subcore_barrier()

        def body(i, carry):
            c0 = RING * i
            for b in range(RING):
                gather(c0 + b, b).wait()
                scatter(c0 + b, b).start(add=True)
            for b in range(LEFT):
                scatter(c0 + b, b).wait()
                gather(c0 + b + RING, b).start()

            @pl.when(i < RITER - 1)
            def _more():
                for b in range(LEFT, RING):
                    scatter(c0 + b, b).wait()
                    gather(c0 + b + RING, b).start()

            return carry

        lax.fori_loop(0, RITER, body, 0)
        cbase = RITER * RING
        rem_g.wait()
        pltpu.sync_copy(rows_r, acc.at[dstr_v], add=True)
        for b in range(LEFT, RING):
            scatter(cbase - RING + b, b).wait()
        plsc.subcore_barrier()
        pltpu.sync_copy(acc.at[pl.ds(sid * ZROWS, ZROWS)],
                        out_hbm.at[cid, pl.ds(sid * ZROWS, ZROWS)])

    return k(y, ei, zeros)


def _tc_proj(x, W):
    def body(x_ref, w_ref, y_ref):
        y_ref[...] = jnp.dot(x_ref[...], w_ref[...],
                             preferred_element_type=jnp.float32)

    return pl.pallas_call(
        body,
        out_shape=jax.ShapeDtypeStruct((N, H), jnp.float32),
    )(x, W)


def _rsqrt(a):
    r = lax.rsqrt(a)
    r = r * (1.5 - 0.5 * a * r * r)
    r = r * (1.5 - 0.5 * a * r * r)
    return r


def _post(h, g, be):
    m = jnp.mean(h, axis=0, keepdims=True)
    v = jnp.mean((h - m) ** 2, axis=0, keepdims=True)
    h = (h - m) * _rsqrt(v + EPS) * g + be
    h = jnp.where(h >= 0, h, 0.01 * h)
    s = jnp.maximum(jnp.sum(h * h, axis=-1, keepdims=True), 1e-24)
    return h * _rsqrt(s)


def _tc_mid(p, r, bl, g, be, Wl):
    def body(p_ref, r_ref, bl_ref, g_ref, be_ref, wl_ref, y_ref, h_ref):
        h = p_ref[0, :N] + p_ref[1, :N] + r_ref[...] + bl_ref[...]
        h = _post(h, g_ref[...], be_ref[...])
        y_ref[...] = jnp.dot(h, wl_ref[...], preferred_element_type=jnp.float32)
        h_ref[...] = h

    return pl.pallas_call(
        body,
        out_shape=[jax.ShapeDtypeStruct((N, H), jnp.float32),
                   jax.ShapeDtypeStruct((N, H), jnp.float32)],
    )(p, r, bl, g, be, Wl)


def _tc_out(p, r, bl, g, be, Wp, bp):
    def body(p_ref, r_ref, bl_ref, g_ref, be_ref, wp_ref, bp_ref, o_ref):
        h = p_ref[0, :N] + p_ref[1, :N] + r_ref[...] + bl_ref[...]
        h = _post(h, g_ref[...], be_ref[...])
        o_ref[...] = (jnp.dot(h, wp_ref[...], preferred_element_type=jnp.float32)
                      + bp_ref[...])

    return pl.pallas_call(
        body,
        out_shape=jax.ShapeDtypeStruct((N, H), jnp.float32),
    )(p, r, bl, g, be, Wp, bp)


def kernel(x, edge_index, W1l, b1l, W1r, g1, be1, W2l, b2l, W2r, g2, be2, Wp, bp):
    zeros = jnp.zeros((NA, H), jnp.float32)

    b1l_, g1_, be1_ = b1l.reshape(1, H), g1.reshape(1, H), be1.reshape(1, H)
    b2l_, g2_, be2_ = b2l.reshape(1, H), g2.reshape(1, H), be2.reshape(1, H)
    bp_ = bp.reshape(1, H)

    y1 = _tc_proj(x, W1l)
    p1 = _sc_segment_sum(y1, edge_index, zeros)
    r1 = _tc_proj(x, W1r)
    y2, h1 = _tc_mid(p1, r1, b1l_, g1_, be1_, W2l)
    p2 = _sc_segment_sum(y2, edge_index, zeros)
    r2 = _tc_proj(h1, W2r)
    return _tc_out(p2, r2, b2l_, g2_, be2_, Wp, bp_)


# speedup vs baseline: 1.4986x; 1.0183x over previous
import functools

import jax
import jax.numpy as jnp
from jax import lax
from jax.experimental import pallas as pl
from jax.experimental.pallas import tpu as pltpu
from jax.experimental.pallas import tpu_sc as plsc

N = 10000
E = 320000
D = 128
H = 32
EPS = 1e-5

NC = 2
NS = 16
NW = NC * NS
EPW = E // NW
CHUNK = 128
NFULL = EPW // CHUNK
REM = EPW - NFULL * CHUNK
RING = 13
RITER = NFULL // RING
LEFT = NFULL - RITER * RING
NA = 10112
ZROWS = NA // NS


def _sc_segment_sum(y, ei, zeros):
    mesh = plsc.VectorSubcoreMesh(core_axis_name="c", subcore_axis_name="s")

    @functools.partial(
        pl.kernel,
        mesh=mesh,
        out_type=jax.ShapeDtypeStruct((NC, NA, H), jnp.float32),
        compiler_params=pltpu.CompilerParams(use_tc_tiling_on_sc=False),
        scratch_types=[
            pltpu.VMEM((EPW,), jnp.int32),
            pltpu.VMEM((NFULL, CHUNK), jnp.int32),
            pltpu.VMEM((REM,), jnp.int32),
            [pltpu.VMEM((CHUNK, H), jnp.float32)] * RING,
            pltpu.VMEM((REM, H), jnp.float32),
            pltpu.VMEM_SHARED((NA, H), jnp.float32),
            pltpu.SemaphoreType.DMA,
            pltpu.SemaphoreType.DMA,
            pltpu.SemaphoreType.DMA,
            [pltpu.SemaphoreType.DMA] * RING,
            [pltpu.SemaphoreType.DMA] * RING,
        ],
    )
    def k(y_hbm, ei_hbm, z_hbm, out_hbm, src_v, dst_v, dstr_v,
          rows, rows_r, acc, semz, semi, semd, sem_g, sem_s):
        cid = lax.axis_index("c")
        sid = lax.axis_index("s")
        wid = cid * NS + sid
        wb = wid * EPW
        zdesc = pltpu.make_async_copy(z_hbm.at[pl.ds(sid * ZROWS, ZROWS)],
                                      acc.at[pl.ds(sid * ZROWS, ZROWS)], semz)
        zdesc.start()
        sdesc = pltpu.make_async_copy(ei_hbm.at[0, pl.ds(wb, EPW)], src_v, semi)
        sdesc.start()

        def dstage(c, carry):
            pltpu.async_copy(ei_hbm.at[1, pl.ds(wb + c * CHUNK, CHUNK)],
                             dst_v.at[c], semd)
            return carry

        lax.fori_loop(0, NFULL, dstage, 0)
        rdesc = pltpu.make_async_copy(
            ei_hbm.at[1, pl.ds(wb + NFULL * CHUNK, REM)], dstr_v, semd)
        rdesc.start()

        def gather(c, b):
            return pltpu.make_async_copy(
                y_hbm.at[src_v.at[pl.ds(c * CHUNK, CHUNK)]], rows[b], sem_g[b])

        def scatter(c, b):
            return pltpu.make_async_copy(rows[b], acc.at[dst_v.at[c]], sem_s[b])

        sdesc.wait()
        for b in range(RING):
            gather(b, b).start()
        rem_g = pltpu.make_async_copy(
            y_hbm.at[src_v.at[pl.ds(NFULL * CHUNK, REM)]], rows_r, semi)
        rem_g.start()

        def dwait(c, carry):
            pltpu.make_async_copy(ei_hbm.at[1, pl.ds(wb + c * CHUNK, CHUNK)],
                                  dst_v.at[c], semd).wait()
            return carry

        lax.fori_loop(0, NFULL, dwait, 0)
        rdesc.wait()
        zdesc.wait()
        plsc.subcore_barrier()

        def body(i, carry):
            c0 = RING * i
            for b in range(RING):
                gather(c0 + b, b).wait()
                scatter(c0 + b, b).start(add=True)
            for b in range(LEFT):
                scatter(c0 + b, b).wait()
                gather(c0 + b + RING, b).start()

            @pl.when(i < RITER - 1)
            def _more():
                for b in range(LEFT, RING):
                    scatter(c0 + b, b).wait()
                    gather(c0 + b + RING, b).start()

            return carry

        lax.fori_loop(0, RITER, body, 0)
        cbase = RITER * RING
        rem_g.wait()
        pltpu.sync_copy(rows_r, acc.at[dstr_v], add=True)
        for b in range(LEFT, RING):
            scatter(cbase - RING + b, b).wait()
        plsc.subcore_barrier()
        pltpu.sync_copy(acc.at[pl.ds(sid * ZROWS, ZROWS)],
                        out_hbm.at[cid, pl.ds(sid * ZROWS, ZROWS)])

    return k(y, ei, zeros)


def _tc_proj(x, W):
    def body(x_ref, w_ref, y_ref):
        y_ref[...] = jnp.dot(x_ref[...], w_ref[...],
                             preferred_element_type=jnp.float32)

    return pl.pallas_call(
        body,
        out_shape=jax.ShapeDtypeStruct((N, H), jnp.float32),
    )(x, W)


def _rsqrt(a):
    r = lax.rsqrt(a)
    r = r * (1.5 - 0.5 * a * r * r)
    r = r * (1.5 - 0.5 * a * r * r)
    return r


def _post(h, g, be):
    m = jnp.mean(h, axis=0, keepdims=True)
    v = jnp.mean((h - m) ** 2, axis=0, keepdims=True)
    h = (h - m) * _rsqrt(v + EPS) * g + be
    h = jnp.where(h >= 0, h, 0.01 * h)
    s = jnp.maximum(jnp.sum(h * h, axis=-1, keepdims=True), 1e-24)
    return h * _rsqrt(s)


def _tc_mid(p, r, bl, g, be, Wl):
    def body(p_ref, r_ref, bl_ref, g_ref, be_ref, wl_ref, y_ref, h_ref):
        h = p_ref[0, :N] + p_ref[1, :N] + r_ref[...] + bl_ref[...]
        h = _post(h, g_ref[...], be_ref[...])
        y_ref[...] = jnp.dot(h, wl_ref[...], preferred_element_type=jnp.float32)
        h_ref[...] = h

    return pl.pallas_call(
        body,
        out_shape=[jax.ShapeDtypeStruct((N, H), jnp.float32),
                   jax.ShapeDtypeStruct((N, H), jnp.float32)],
    )(p, r, bl, g, be, Wl)


def _tc_out(p, r, bl, g, be, Wp, bp):
    def body(p_ref, r_ref, bl_ref, g_ref, be_ref, wp_ref, bp_ref, o_ref):
        h = p_ref[0, :N] + p_ref[1, :N] + r_ref[...] + bl_ref[...]
        h = _post(h, g_ref[...], be_ref[...])
        o_ref[...] = (jnp.dot(h, wp_ref[...], preferred_element_type=jnp.float32)
                      + bp_ref[...])

    return pl.pallas_call(
        body,
        out_shape=jax.ShapeDtypeStruct((N, H), jnp.float32),
    )(p, r, bl, g, be, Wp, bp)


def kernel(x, edge_index, W1l, b1l, W1r, g1, be1, W2l, b2l, W2r, g2, be2, Wp, bp):
    zeros = jnp.zeros((NA, H), jnp.float32)

    b1l_, g1_, be1_ = b1l.reshape(1, H), g1.reshape(1, H), be1.reshape(1, H)
    b2l_, g2_, be2_ = b2l.reshape(1, H), g2.reshape(1, H), be2.reshape(1, H)
    bp_ = bp.reshape(1, H)

    y1 = _tc_proj(x, W1l)
    p1 = _sc_segment_sum(y1, edge_index, zeros)
    r1 = _tc_proj(x, W1r)
    y2, h1 = _tc_mid(p1, r1, b1l_, g1_, be1_, W2l)
    p2 = _sc_segment_sum(y2, edge_index, zeros)
    r2 = _tc_proj(h1, W2r)
    return _tc_out(p2, r2, b2l_, g2_, be2_, Wp, bp_)
